# baseline (speedup 1.0000x reference)
import dataclasses
import functools

import jax
import jax.numpy as jnp
from jax import lax
from jax.experimental import pallas as pl
from jax.experimental.pallas import tpu as pltpu
from jax.experimental.pallas import tpu_sc as plsc

N = 10000
E = 320000
D = 128
NC = 2
NS = 16
NW = NC * NS
EPT = E // NW
CH = 80
NCHUNK = EPT // CH
NGRP = 5
GCH = NCHUNK // NGRP
RPT = N // NS

_MESH = plsc.VectorSubcoreMesh(core_axis_name="c", subcore_axis_name="s")

_SC_PARAMS = pltpu.CompilerParams()
if "needs_layout_passes" in pltpu.CompilerParams.__dataclass_fields__:
    _SC_PARAMS = dataclasses.replace(_SC_PARAMS, needs_layout_passes=False)


def _copy_out_stripe(acc_sh, out_hbm, c, s):
    pltpu.sync_copy(acc_sh.at[pl.ds(s * RPT, RPT)], out_hbm.at[c, s])


@functools.partial(
    pl.kernel,
    out_type=jax.ShapeDtypeStruct((NC, NS, RPT, D), jnp.float32),
    mesh=_MESH,
    scratch_types=[
        pltpu.VMEM((GCH, CH), jnp.int32),
        pltpu.VMEM((GCH, CH), jnp.float32),
        pltpu.VMEM((CH, D), jnp.float32),
        pltpu.VMEM_SHARED((N, D), jnp.float32),
    ],
    compiler_params=_SC_PARAMS,
)
def _sc_deg(col_hbm, w_hbm, out_hbm, col_v, w_v, msg_v, acc_sh):
    c = lax.axis_index("c")
    s = lax.axis_index("s")
    wid = c * NS + s

    @pl.loop(0, CH)
    def _(r):
        for i in range(D // 16):
            msg_v[r, pl.ds(i * 16, 16)] = jnp.zeros((16,), jnp.float32)

    @pl.loop(0, RPT // CH)
    def _(q):
        pltpu.sync_copy(msg_v, acc_sh.at[pl.ds(s * RPT + q * CH, CH)])
    pltpu.sync_copy(msg_v.at[pl.ds(0, RPT % CH)],
                    acc_sh.at[pl.ds(s * RPT + (RPT // CH) * CH, RPT % CH)])

    plsc.subcore_barrier()

    e0 = jnp.where(lax.iota(jnp.int32, 16) == 0,
                   jnp.float32(1.0), jnp.float32(0.0))

    @pl.loop(0, NGRP)
    def _(g):
        pltpu.sync_copy(col_hbm.at[wid, g], col_v)
        pltpu.sync_copy(w_hbm.at[wid, g], w_v)

        @pl.loop(0, GCH)
        def _(j):
            for kk in range(0, CH, 16):
                wv = w_v[j, pl.ds(kk, 16)]
                for l in range(16):
                    msg_v[kk + l, pl.ds(0, 16)] = (
                        jnp.full((16,), wv[l], jnp.float32) * e0)
            pltpu.sync_copy(msg_v, acc_sh.at[col_v.at[j]], add=True)

    plsc.subcore_barrier()
    _copy_out_stripe(acc_sh, out_hbm, c, s)


@functools.partial(
    pl.kernel,
    out_type=jax.ShapeDtypeStruct((NC, NS, RPT, D), jnp.float32),
    mesh=_MESH,
    scratch_types=[
        pltpu.VMEM((GCH, CH), jnp.int32),
        pltpu.VMEM((GCH, CH), jnp.int32),
        pltpu.VMEM((GCH, CH), jnp.float32),
        pltpu.VMEM((CH, D), jnp.float32),
        pltpu.VMEM((CH, D), jnp.float32),
        pltpu.VMEM_SHARED((N, D), jnp.float32),
        pltpu.SemaphoreType.DMA,
        pltpu.SemaphoreType.DMA,
        pltpu.SemaphoreType.DMA,
        pltpu.SemaphoreType.DMA,
    ],
    compiler_params=_SC_PARAMS,
)
def _sc_msg(y_hbm, row_hbm, col_hbm, w_hbm, out_hbm,
            row_v, col_v, w_v, msg_a, msg_b, acc_sh, gsa, gsb, ssa, ssb):
    c = lax.axis_index("c")
    s = lax.axis_index("s")
    wid = c * NS + s

    @pl.loop(0, CH)
    def _(r):
        for i in range(D // 16):
            msg_a[r, pl.ds(i * 16, 16)] = jnp.zeros((16,), jnp.float32)

    @pl.loop(0, RPT // CH)
    def _(q):
        pltpu.sync_copy(msg_a, acc_sh.at[pl.ds(s * RPT + q * CH, CH)])
    pltpu.sync_copy(msg_a.at[pl.ds(0, RPT % CH)],
                    acc_sh.at[pl.ds(s * RPT + (RPT // CH) * CH, RPT % CH)])

    plsc.subcore_barrier()

    def _scale(msg_v, j):
        @plsc.parallel_loop(0, CH, step=16)
        def _(kk):
            wv = w_v[j, pl.ds(kk, 16)]
            for l in range(16):
                ws = jnp.full((16,), wv[l], jnp.float32)
                for i in range(D // 16):
                    sl = pl.ds(i * 16, 16)
                    msg_v[kk + l, sl] = msg_v[kk + l, sl] * ws

    def _g_start(buf, sem, j):
        pltpu.async_copy(y_hbm.at[row_v.at[j]], buf, sem)

    def _g_drain(buf, sem, j):
        pltpu.make_async_copy(y_hbm.at[row_v.at[j]], buf, sem).wait()

    def _s_start(buf, sem, j):
        pltpu.async_copy(buf, acc_sh.at[col_v.at[j]], sem, add=True)

    def _s_drain(buf, sem, j):
        pltpu.make_async_copy(buf, acc_sh.at[col_v.at[j]], sem).wait()

    @pl.loop(0, NGRP)
    def _(g):
        pltpu.sync_copy(row_hbm.at[wid, g], row_v)
        pltpu.sync_copy(col_hbm.at[wid, g], col_v)
        pltpu.sync_copy(w_hbm.at[wid, g], w_v)

        _g_start(msg_a, gsa, 0)
        _g_start(msg_b, gsb, 1)

        @pl.loop(0, GCH - 1, step=2)
        def _(j):
            _g_drain(msg_a, gsa, j)
            _scale(msg_a, j)
            _s_start(msg_a, ssa, j)
            _g_drain(msg_b, gsb, j + 1)
            _scale(msg_b, j + 1)
            _s_start(msg_b, ssb, j + 1)
            _s_drain(msg_a, ssa, j)
            _g_start(msg_a, gsa, j + 2)
            _s_drain(msg_b, ssb, j + 1)

            @pl.when(j < GCH - 3)
            def _():
                _g_start(msg_b, gsb, j + 3)

        _g_drain(msg_a, gsa, GCH - 1)
        _scale(msg_a, GCH - 1)
        pltpu.sync_copy(msg_a, acc_sh.at[col_v.at[GCH - 1]], add=True)

    plsc.subcore_barrier()
    _copy_out_stripe(acc_sh, out_hbm, c, s)


_BLK = 1000
_GRID = N // _BLK


def _tc_matmul(x, W):
    def body(x_ref, w_ref, o_ref):
        o_ref[...] = lax.dot_general(
            x_ref[...], w_ref[...], (((1,), (0,)), ((), ())),
            precision=lax.Precision.HIGHEST,
            preferred_element_type=jnp.float32)

    return pl.pallas_call(
        body,
        grid=(_GRID,),
        in_specs=[pl.BlockSpec((_BLK, D), lambda i: (i, 0)),
                  pl.BlockSpec((D, D), lambda i: (0, 0))],
        out_specs=pl.BlockSpec((_BLK, D), lambda i: (i, 0)),
        out_shape=jax.ShapeDtypeStruct((N, D), jnp.float32),
    )(x, W)


def _dinv_of(deg_ref):
    d = jnp.sum(deg_ref[...], axis=(0, 2)) + 1.0
    return jnp.where(d > 0, lax.rsqrt(d), 0.0)


_DEG_SPEC = pl.BlockSpec((NC, _BLK, D), lambda i: (0, i, 0))


def _tc_prep(deg16, xw):
    def body(deg_ref, xw_ref, y_ref):
        y_ref[...] = xw_ref[...] * _dinv_of(deg_ref)[:, None]

    return pl.pallas_call(
        body,
        grid=(_GRID,),
        in_specs=[_DEG_SPEC,
                  pl.BlockSpec((_BLK, D), lambda i: (i, 0))],
        out_specs=pl.BlockSpec((_BLK, D), lambda i: (i, 0)),
        out_shape=jax.ShapeDtypeStruct((N, D), jnp.float32),
    )(deg16, xw)


def _combine(acc_ref, xw_ref, dinv, b_ref):
    pre = dinv[:, None] * (acc_ref[0] + acc_ref[1] + dinv[:, None] * xw_ref[...])
    return jnp.maximum(pre + b_ref[...][None, :], 0.0)


def _tc_mid(acc, xw1, deg16, b1, W2):
    def body(acc_ref, xw_ref, deg_ref, b_ref, w2_ref, xw2_ref, y2_ref):
        dinv = _dinv_of(deg_ref)
        h = _combine(acc_ref, xw_ref, dinv, b_ref)
        xw2 = lax.dot_general(
            h, w2_ref[...], (((1,), (0,)), ((), ())),
            precision=lax.Precision.HIGHEST,
            preferred_element_type=jnp.float32)
        xw2_ref[...] = xw2
        y2_ref[...] = xw2 * dinv[:, None]

    return pl.pallas_call(
        body,
        grid=(_GRID,),
        in_specs=[pl.BlockSpec((NC, _BLK, D), lambda i: (0, i, 0)),
                  pl.BlockSpec((_BLK, D), lambda i: (i, 0)),
                  _DEG_SPEC,
                  pl.BlockSpec((D,), lambda i: (0,)),
                  pl.BlockSpec((D, D), lambda i: (0, 0))],
        out_specs=[pl.BlockSpec((_BLK, D), lambda i: (i, 0)),
                   pl.BlockSpec((_BLK, D), lambda i: (i, 0))],
        out_shape=[jax.ShapeDtypeStruct((N, D), jnp.float32),
                   jax.ShapeDtypeStruct((N, D), jnp.float32)],
    )(acc, xw1, deg16, b1, W2)


def _tc_post(acc, xw2, deg16, b2):
    def body(acc_ref, xw_ref, deg_ref, b_ref, o_ref):
        o_ref[...] = _combine(acc_ref, xw_ref, _dinv_of(deg_ref), b_ref)

    return pl.pallas_call(
        body,
        grid=(_GRID,),
        in_specs=[pl.BlockSpec((NC, _BLK, D), lambda i: (0, i, 0)),
                  pl.BlockSpec((_BLK, D), lambda i: (i, 0)),
                  _DEG_SPEC,
                  pl.BlockSpec((D,), lambda i: (0,))],
        out_specs=pl.BlockSpec((_BLK, D), lambda i: (i, 0)),
        out_shape=jax.ShapeDtypeStruct((N, D), jnp.float32),
    )(acc, xw2, deg16, b2)


def kernel(x, edge_index, edge_weight, W1, b1, W2, b2):
    row = edge_index[0].astype(jnp.int32)
    col = edge_index[1].astype(jnp.int32)
    row4 = row.reshape(NW, NGRP, GCH, CH)
    col4 = col.reshape(NW, NGRP, GCH, CH)
    w4 = edge_weight.reshape(NW, NGRP, GCH, CH)
    deg16 = _sc_deg(col4, w4).reshape(NC, N, D)
    xw1 = _tc_matmul(x, W1)
    y1 = _tc_prep(deg16, xw1)

    acc1 = _sc_msg(y1, row4, col4, w4).reshape(NC, N, D)
    xw2, y2 = _tc_mid(acc1, xw1, deg16, b1, W2)

    acc2 = _sc_msg(y2, row4, col4, w4).reshape(NC, N, D)
    return _tc_post(acc2, xw2, deg16, b2)


# speedup vs baseline: 1.1420x; 1.1420x over previous
import dataclasses
import functools

import jax
import jax.numpy as jnp
from jax import lax
from jax.experimental import pallas as pl
from jax.experimental.pallas import tpu as pltpu
from jax.experimental.pallas import tpu_sc as plsc

N = 10000
E = 320000
D = 128
NC = 2
NS = 16
NW = NC * NS
EPT = E // NW
CH = 80
NCHUNK = EPT // CH
NGRP = 5
GCH = NCHUNK // NGRP
RPT = N // NS

_MESH = plsc.VectorSubcoreMesh(core_axis_name="c", subcore_axis_name="s")

_SC_PARAMS = pltpu.CompilerParams()
if "needs_layout_passes" in pltpu.CompilerParams.__dataclass_fields__:
    _SC_PARAMS = dataclasses.replace(_SC_PARAMS, needs_layout_passes=False)


def _copy_out_stripe(acc_sh, out_hbm, c, s):
    pltpu.sync_copy(acc_sh.at[pl.ds(s * RPT, RPT)], out_hbm.at[c, s])


NPAD = 10240
SPT = NPAD // NS


@functools.partial(
    pl.kernel,
    out_type=jax.ShapeDtypeStruct((NC, NS, SPT, D), jnp.float32),
    mesh=_MESH,
    scratch_types=[
        pltpu.VMEM((GCH, CH), jnp.int32),
        pltpu.VMEM((GCH, CH), jnp.float32),
        pltpu.VMEM((NPAD,), jnp.float32),
        pltpu.VMEM((NS, SPT), jnp.float32),
        pltpu.VMEM((16, D), jnp.float32),
        pltpu.VMEM_SHARED((NS, NPAD), jnp.float32),
    ],
    compiler_params=_SC_PARAMS,
)
def _sc_deg(col_hbm, w_hbm, out_hbm, col_v, w_v, hist_v, red_v, stg_v, hist_sh):
    c = lax.axis_index("c")
    s = lax.axis_index("s")
    wid = c * NS + s

    @pl.loop(0, NPAD, step=16)
    def _(r):
        hist_v[pl.ds(r, 16)] = jnp.zeros((16,), jnp.float32)

    @pl.loop(0, 16)
    def _(r):
        for i in range(D // 16):
            stg_v[r, pl.ds(i * 16, 16)] = jnp.zeros((16,), jnp.float32)

    @pl.loop(0, NGRP)
    def _(g):
        pltpu.sync_copy(col_hbm.at[wid, g], col_v)
        pltpu.sync_copy(w_hbm.at[wid, g], w_v)

        @pl.loop(0, GCH)
        def _(j):
            for kk in range(0, CH, 16):
                cv = col_v[j, pl.ds(kk, 16)]
                wv = w_v[j, pl.ds(kk, 16)]
                plsc.addupdate_scatter(hist_v, [cv], wv)

    pltpu.sync_copy(hist_v, hist_sh.at[s])
    plsc.subcore_barrier()

    pltpu.sync_copy(hist_sh.at[:, pl.ds(s * SPT, SPT)], red_v)
    iota16 = lax.iota(jnp.int32, 16)
    zid = jnp.zeros((16,), jnp.int32)

    @pl.loop(0, SPT, step=16)
    def _(q):
        acc = red_v[0, pl.ds(q, 16)]
        for t in range(1, NS):
            acc = acc + red_v[t, pl.ds(q, 16)]
        plsc.store_scatter(stg_v, [iota16, zid], acc)
        pltpu.sync_copy(stg_v, out_hbm.at[c, s, pl.ds(q, 16)])


@functools.partial(
    pl.kernel,
    out_type=jax.ShapeDtypeStruct((NC, NS, RPT, D), jnp.float32),
    mesh=_MESH,
    scratch_types=[
        pltpu.VMEM((GCH, CH), jnp.int32),
        pltpu.VMEM((GCH, CH), jnp.int32),
        pltpu.VMEM((GCH, CH), jnp.float32),
        pltpu.VMEM((CH, D), jnp.float32),
        pltpu.VMEM((CH, D), jnp.float32),
        pltpu.VMEM_SHARED((N, D), jnp.float32),
        pltpu.SemaphoreType.DMA,
        pltpu.SemaphoreType.DMA,
        pltpu.SemaphoreType.DMA,
        pltpu.SemaphoreType.DMA,
    ],
    compiler_params=_SC_PARAMS,
)
def _sc_msg(y_hbm, row_hbm, col_hbm, w_hbm, out_hbm,
            row_v, col_v, w_v, msg_a, msg_b, acc_sh, gsa, gsb, ssa, ssb):
    c = lax.axis_index("c")
    s = lax.axis_index("s")
    wid = c * NS + s

    @pl.loop(0, CH)
    def _(r):
        for i in range(D // 16):
            msg_a[r, pl.ds(i * 16, 16)] = jnp.zeros((16,), jnp.float32)

    @pl.loop(0, RPT // CH)
    def _(q):
        pltpu.sync_copy(msg_a, acc_sh.at[pl.ds(s * RPT + q * CH, CH)])
    pltpu.sync_copy(msg_a.at[pl.ds(0, RPT % CH)],
                    acc_sh.at[pl.ds(s * RPT + (RPT // CH) * CH, RPT % CH)])

    plsc.subcore_barrier()

    def _scale(msg_v, j):
        @pl.loop(0, CH, step=16)
        def _(kk):
            wv = w_v[j, pl.ds(kk, 16)]
            for l in range(16):
                ws = jnp.full((16,), wv[l], jnp.float32)
                for i in range(D // 16):
                    sl = pl.ds(i * 16, 16)
                    msg_v[kk + l, sl] = msg_v[kk + l, sl] * ws

    def _g_start(buf, sem, j):
        pltpu.async_copy(y_hbm.at[row_v.at[j]], buf, sem)

    def _g_drain(buf, sem, j):
        pltpu.make_async_copy(y_hbm.at[row_v.at[j]], buf, sem).wait()

    def _s_start(buf, sem, j):
        pltpu.async_copy(buf, acc_sh.at[col_v.at[j]], sem, add=True)

    def _s_drain(buf, sem, j):
        pltpu.make_async_copy(buf, acc_sh.at[col_v.at[j]], sem).wait()

    @pl.loop(0, NGRP)
    def _(g):
        pltpu.sync_copy(row_hbm.at[wid, g], row_v)
        pltpu.sync_copy(col_hbm.at[wid, g], col_v)
        pltpu.sync_copy(w_hbm.at[wid, g], w_v)

        _g_start(msg_a, gsa, 0)
        _g_start(msg_b, gsb, 1)

        @pl.loop(0, GCH - 1, step=2)
        def _(j):
            _g_drain(msg_a, gsa, j)
            _scale(msg_a, j)
            _s_start(msg_a, ssa, j)
            _g_drain(msg_b, gsb, j + 1)
            _scale(msg_b, j + 1)
            _s_start(msg_b, ssb, j + 1)
            _s_drain(msg_a, ssa, j)
            _g_start(msg_a, gsa, j + 2)
            _s_drain(msg_b, ssb, j + 1)

            @pl.when(j < GCH - 3)
            def _():
                _g_start(msg_b, gsb, j + 3)

        _g_drain(msg_a, gsa, GCH - 1)
        _scale(msg_a, GCH - 1)
        pltpu.sync_copy(msg_a, acc_sh.at[col_v.at[GCH - 1]], add=True)

    plsc.subcore_barrier()
    _copy_out_stripe(acc_sh, out_hbm, c, s)


_BLK = 1000
_GRID = N // _BLK


def _tc_matmul(x, W):
    def body(x_ref, w_ref, o_ref):
        o_ref[...] = lax.dot_general(
            x_ref[...], w_ref[...], (((1,), (0,)), ((), ())),
            precision=lax.Precision.HIGHEST,
            preferred_element_type=jnp.float32)

    return pl.pallas_call(
        body,
        grid=(_GRID,),
        in_specs=[pl.BlockSpec((_BLK, D), lambda i: (i, 0)),
                  pl.BlockSpec((D, D), lambda i: (0, 0))],
        out_specs=pl.BlockSpec((_BLK, D), lambda i: (i, 0)),
        out_shape=jax.ShapeDtypeStruct((N, D), jnp.float32),
    )(x, W)


def _dinv_of(deg_ref):
    d = jnp.sum(deg_ref[...], axis=(0, 2)) + 1.0
    return jnp.where(d > 0, lax.rsqrt(d), 0.0)


_DEG_SPEC = pl.BlockSpec((NC, _BLK, D), lambda i: (0, i, 0))


def _tc_prep(deg16, xw):
    def body(deg_ref, xw_ref, y_ref):
        y_ref[...] = xw_ref[...] * _dinv_of(deg_ref)[:, None]

    return pl.pallas_call(
        body,
        grid=(_GRID,),
        in_specs=[_DEG_SPEC,
                  pl.BlockSpec((_BLK, D), lambda i: (i, 0))],
        out_specs=pl.BlockSpec((_BLK, D), lambda i: (i, 0)),
        out_shape=jax.ShapeDtypeStruct((N, D), jnp.float32),
    )(deg16, xw)


def _combine(acc_ref, xw_ref, dinv, b_ref):
    pre = dinv[:, None] * (acc_ref[0] + acc_ref[1] + dinv[:, None] * xw_ref[...])
    return jnp.maximum(pre + b_ref[...][None, :], 0.0)


def _tc_mid(acc, xw1, deg16, b1, W2):
    def body(acc_ref, xw_ref, deg_ref, b_ref, w2_ref, xw2_ref, y2_ref):
        dinv = _dinv_of(deg_ref)
        h = _combine(acc_ref, xw_ref, dinv, b_ref)
        xw2 = lax.dot_general(
            h, w2_ref[...], (((1,), (0,)), ((), ())),
            precision=lax.Precision.HIGHEST,
            preferred_element_type=jnp.float32)
        xw2_ref[...] = xw2
        y2_ref[...] = xw2 * dinv[:, None]

    return pl.pallas_call(
        body,
        grid=(_GRID,),
        in_specs=[pl.BlockSpec((NC, _BLK, D), lambda i: (0, i, 0)),
                  pl.BlockSpec((_BLK, D), lambda i: (i, 0)),
                  _DEG_SPEC,
                  pl.BlockSpec((D,), lambda i: (0,)),
                  pl.BlockSpec((D, D), lambda i: (0, 0))],
        out_specs=[pl.BlockSpec((_BLK, D), lambda i: (i, 0)),
                   pl.BlockSpec((_BLK, D), lambda i: (i, 0))],
        out_shape=[jax.ShapeDtypeStruct((N, D), jnp.float32),
                   jax.ShapeDtypeStruct((N, D), jnp.float32)],
    )(acc, xw1, deg16, b1, W2)


def _tc_post(acc, xw2, deg16, b2):
    def body(acc_ref, xw_ref, deg_ref, b_ref, o_ref):
        o_ref[...] = _combine(acc_ref, xw_ref, _dinv_of(deg_ref), b_ref)

    return pl.pallas_call(
        body,
        grid=(_GRID,),
        in_specs=[pl.BlockSpec((NC, _BLK, D), lambda i: (0, i, 0)),
                  pl.BlockSpec((_BLK, D), lambda i: (i, 0)),
                  _DEG_SPEC,
                  pl.BlockSpec((D,), lambda i: (0,))],
        out_specs=pl.BlockSpec((_BLK, D), lambda i: (i, 0)),
        out_shape=jax.ShapeDtypeStruct((N, D), jnp.float32),
    )(acc, xw2, deg16, b2)


def kernel(x, edge_index, edge_weight, W1, b1, W2, b2):
    row = edge_index[0].astype(jnp.int32)
    col = edge_index[1].astype(jnp.int32)
    row4 = row.reshape(NW, NGRP, GCH, CH)
    col4 = col.reshape(NW, NGRP, GCH, CH)
    w4 = edge_weight.reshape(NW, NGRP, GCH, CH)
    deg16 = _sc_deg(col4, w4).reshape(NC, NPAD, D)[:, :N, :]
    xw1 = _tc_matmul(x, W1)
    y1 = _tc_prep(deg16, xw1)

    acc1 = _sc_msg(y1, row4, col4, w4).reshape(NC, N, D)
    xw2, y2 = _tc_mid(acc1, xw1, deg16, b1, W2)

    acc2 = _sc_msg(y2, row4, col4, w4).reshape(NC, N, D)
    return _tc_post(acc2, xw2, deg16, b2)
